# baseline (speedup 1.0000x reference)
import jax
import jax.numpy as jnp
import numpy as np
from jax import lax
from jax.experimental import pallas as pl
from jax.experimental.pallas import tpu as pltpu

N_DIRS = 24576
D_MODEL = 768
K = 64
AUXK = 256
DEAD_STEPS_THRESHOLD = 266
TOKENS = 2048

MINT = np.int32(-2147483648)


ENC_N_TILE = 1024
ENC_T_TILE = 512


def _enc_body(theta_ref, x_ref, w_ref, b_ref, lpa_ref, jump_ref):
    acc = jax.lax.dot_general(
        x_ref[...], w_ref[...], (((1,), (1,)), ((), ())),
        preferred_element_type=jnp.float32,
    )
    lpa = acc + b_ref[...]
    lpa_ref[...] = lpa
    theta = theta_ref[0]
    jump_ref[...] = jnp.where(lpa < theta, jnp.zeros_like(lpa), lpa)


def _encoder(xc, W_enc, latent_bias, theta):
    grid = (N_DIRS // ENC_N_TILE, TOKENS // ENC_T_TILE)
    return pl.pallas_call(
        _enc_body,
        grid=grid,
        in_specs=[
            pl.BlockSpec(memory_space=pltpu.SMEM),
            pl.BlockSpec((ENC_T_TILE, D_MODEL), lambda n, t: (t, 0)),
            pl.BlockSpec((ENC_N_TILE, D_MODEL), lambda n, t: (n, 0)),
            pl.BlockSpec((1, ENC_N_TILE), lambda n, t: (0, n)),
        ],
        out_specs=[
            pl.BlockSpec((ENC_T_TILE, ENC_N_TILE), lambda n, t: (t, n)),
            pl.BlockSpec((ENC_T_TILE, ENC_N_TILE), lambda n, t: (t, n)),
        ],
        out_shape=[
            jax.ShapeDtypeStruct((TOKENS, N_DIRS), jnp.float32),
            jax.ShapeDtypeStruct((TOKENS, N_DIRS), jnp.float32),
        ],
    )(theta.reshape(1), xc, W_enc, latent_bias.reshape(1, N_DIRS))


def _skey(v):
    bits = jax.lax.bitcast_convert_type(v, jnp.int32)
    return jnp.where(bits >= 0, bits, bits ^ np.int32(0x7FFFFFFF))


def _skey_to_float(sk):
    bits = jnp.where(sk >= 0, sk, sk ^ np.int32(0x7FFFFFFF))
    return jax.lax.bitcast_convert_type(bits, jnp.float32)


def _search_kth(skeys, kth):
    rows = skeys.shape[0]
    ut0 = jnp.zeros((rows, 1), jnp.int32)

    def body(i, ut):
        b = 31 - i
        cand_u = ut | (np.int32(1) << b)
        cand_s = cand_u ^ MINT
        cnt = jnp.sum((skeys >= cand_s).astype(jnp.int32), axis=1, keepdims=True)
        return jnp.where(cnt >= kth, cand_u, ut)

    ut = jax.lax.fori_loop(0, 32, body, ut0)
    return ut ^ MINT


THR_T_TILE = 128


def _thr_body(lpa_ref, t64_ref, t256_ref, fired_ref):
    skeys = _skey(lpa_ref[...])
    t64 = _search_kth(skeys, K)
    t256 = _search_kth(skeys, 4 * K)
    t64_ref[...] = jnp.broadcast_to(_skey_to_float(t64), t64_ref.shape)
    t256_ref[...] = jnp.broadcast_to(_skey_to_float(t256), t256_ref.shape)
    fired = jnp.max((skeys >= t64).astype(jnp.int32), axis=0, keepdims=True)
    fired_ref[...] = fired.reshape(fired_ref.shape)


def _thresholds(lpa):
    grid = (TOKENS // THR_T_TILE,)
    return pl.pallas_call(
        _thr_body,
        grid=grid,
        in_specs=[pl.BlockSpec((THR_T_TILE, N_DIRS), lambda t: (t, 0))],
        out_specs=[
            pl.BlockSpec((THR_T_TILE, 128), lambda t: (t, 0)),
            pl.BlockSpec((THR_T_TILE, 128), lambda t: (t, 0)),
            pl.BlockSpec((1, 1, N_DIRS), lambda t: (t, 0, 0)),
        ],
        out_shape=[
            jax.ShapeDtypeStruct((TOKENS, 128), jnp.float32),
            jax.ShapeDtypeStruct((TOKENS, 128), jnp.float32),
            jax.ShapeDtypeStruct((TOKENS // THR_T_TILE, 1, N_DIRS), jnp.int32),
        ],
    )(lpa)


def _aux_body(lpa_ref, stats_ref, fired_ref, taux_ref, dead_ref):
    fired = jnp.max(fired_ref[...], axis=(0, 1)).reshape(1, N_DIRS)
    alive = stats_ref[...] + 1 <= DEAD_STEPS_THRESHOLD
    dead = jnp.logical_and(jnp.logical_not(alive), fired == 0)
    skeys = jnp.where(dead, _skey(lpa_ref[...]), jnp.zeros((), jnp.int32))
    taux = _search_kth(skeys, AUXK)
    taux_ref[...] = jnp.broadcast_to(_skey_to_float(taux), taux_ref.shape)
    dead_ref[...] = dead.astype(jnp.float32)


def _aux_thresholds(lpa, stats, fired_part):
    grid = (TOKENS // THR_T_TILE,)
    nfp = fired_part.shape[0]
    return pl.pallas_call(
        _aux_body,
        grid=grid,
        in_specs=[
            pl.BlockSpec((THR_T_TILE, N_DIRS), lambda t: (t, 0)),
            pl.BlockSpec((1, N_DIRS), lambda t: (0, 0)),
            pl.BlockSpec((nfp, 1, N_DIRS), lambda t: (0, 0, 0)),
        ],
        out_specs=[
            pl.BlockSpec((THR_T_TILE, 128), lambda t: (t, 0)),
            pl.BlockSpec((1, N_DIRS), lambda t: (0, 0)),
        ],
        out_shape=[
            jax.ShapeDtypeStruct((TOKENS, 128), jnp.float32),
            jax.ShapeDtypeStruct((1, N_DIRS), jnp.float32),
        ],
    )(lpa, stats.reshape(1, N_DIRS), fired_part)


REC_T_TILE = 512
REC_N_TILE = 2048


def _rec_body(lpa_ref, w_ref, t64_ref, t256_ref, pb_ref, r64_ref, r256_ref,
              acc64, acc256):
    n = pl.program_id(1)
    lpa = lpa_ref[...]
    relu = jnp.maximum(lpa, 0.0)
    t64 = t64_ref[...][:, :1]
    t256 = t256_ref[...][:, :1]
    l64 = jnp.where(lpa >= t64, relu, 0.0)
    l256 = jnp.where(lpa >= t256, relu, 0.0)
    w = w_ref[...]
    p64 = jax.lax.dot_general(l64, w, (((1,), (1,)), ((), ())),
                              preferred_element_type=jnp.float32)
    p256 = jax.lax.dot_general(l256, w, (((1,), (1,)), ((), ())),
                               preferred_element_type=jnp.float32)

    @pl.when(n == 0)
    def _init():
        acc64[...] = p64
        acc256[...] = p256

    @pl.when(n != 0)
    def _acc():
        acc64[...] += p64
        acc256[...] += p256

    @pl.when(n == pl.num_programs(1) - 1)
    def _emit():
        pb = pb_ref[...]
        r64_ref[...] = acc64[...] + pb
        r256_ref[...] = acc256[...] + pb


def _recons(lpa, W_dec, t64f, t256f, pre_bias):
    grid = (TOKENS // REC_T_TILE, N_DIRS // REC_N_TILE)
    return pl.pallas_call(
        _rec_body,
        grid=grid,
        in_specs=[
            pl.BlockSpec((REC_T_TILE, REC_N_TILE), lambda t, n: (t, n)),
            pl.BlockSpec((D_MODEL, REC_N_TILE), lambda t, n: (0, n)),
            pl.BlockSpec((REC_T_TILE, 128), lambda t, n: (t, 0)),
            pl.BlockSpec((REC_T_TILE, 128), lambda t, n: (t, 0)),
            pl.BlockSpec((1, D_MODEL), lambda t, n: (0, 0)),
        ],
        out_specs=[
            pl.BlockSpec((REC_T_TILE, D_MODEL), lambda t, n: (t, 0)),
            pl.BlockSpec((REC_T_TILE, D_MODEL), lambda t, n: (t, 0)),
        ],
        out_shape=[
            jax.ShapeDtypeStruct((TOKENS, D_MODEL), jnp.float32),
            jax.ShapeDtypeStruct((TOKENS, D_MODEL), jnp.float32),
        ],
        scratch_shapes=[
            pltpu.VMEM((REC_T_TILE, D_MODEL), jnp.float32),
            pltpu.VMEM((REC_T_TILE, D_MODEL), jnp.float32),
        ],
    )(lpa, W_dec, t64f, t256f, pre_bias.reshape(1, D_MODEL))


def kernel(x, W_enc, W_dec, pre_bias, latent_bias, stats_last_nonzero, theta):
    xc = x - pre_bias
    theta = jnp.asarray(theta, jnp.float32)
    lpa, latents_jump = _encoder(xc, W_enc, latent_bias, theta)

    t64f, t256f, fired_part = _thresholds(lpa)
    tauxf, dead = _aux_thresholds(lpa, stats_last_nonzero, fired_part)
    recons, multik_recons = _recons(lpa, W_dec, t64f, t256f, pre_bias)

    multik_values, multik_indices = lax.top_k(lpa, 4 * K)
    multik_values = jnp.maximum(multik_values, 0.0)
    topk_indices = multik_indices[:, :K]
    topk_values = multik_values[:, :K]

    auxk_values, auxk_indices = lax.top_k(lpa * dead, AUXK)
    auxk_values = jnp.maximum(auxk_values, 0.0)

    return (recons, topk_indices, topk_values, multik_indices, multik_values,
            multik_recons, auxk_indices, auxk_values, lpa, latents_jump)


# speedup vs baseline: 1.0923x; 1.0923x over previous
import jax
import jax.numpy as jnp
import numpy as np
from jax import lax
from jax.experimental import pallas as pl
from jax.experimental.pallas import tpu as pltpu

N_DIRS = 24576
D_MODEL = 768
K = 64
AUXK = 256
DEAD_STEPS_THRESHOLD = 266
TOKENS = 2048

MINT = np.int32(-2147483648)


ENC_N_TILE = 1024
ENC_T_TILE = 512


def _enc_body(theta_ref, x_ref, w_ref, b_ref, lpa_ref, jump_ref):
    acc = jax.lax.dot_general(
        x_ref[...], w_ref[...], (((1,), (1,)), ((), ())),
        preferred_element_type=jnp.float32,
    )
    lpa = acc + b_ref[...]
    lpa_ref[...] = lpa
    theta = theta_ref[0]
    jump_ref[...] = jnp.where(lpa < theta, jnp.zeros_like(lpa), lpa)


def _encoder(xc, W_enc, latent_bias, theta):
    grid = (N_DIRS // ENC_N_TILE, TOKENS // ENC_T_TILE)
    return pl.pallas_call(
        _enc_body,
        grid=grid,
        in_specs=[
            pl.BlockSpec(memory_space=pltpu.SMEM),
            pl.BlockSpec((ENC_T_TILE, D_MODEL), lambda n, t: (t, 0)),
            pl.BlockSpec((ENC_N_TILE, D_MODEL), lambda n, t: (n, 0)),
            pl.BlockSpec((1, ENC_N_TILE), lambda n, t: (0, n)),
        ],
        out_specs=[
            pl.BlockSpec((ENC_T_TILE, ENC_N_TILE), lambda n, t: (t, n)),
            pl.BlockSpec((ENC_T_TILE, ENC_N_TILE), lambda n, t: (t, n)),
        ],
        out_shape=[
            jax.ShapeDtypeStruct((TOKENS, N_DIRS), jnp.float32),
            jax.ShapeDtypeStruct((TOKENS, N_DIRS), jnp.float32),
        ],
    )(theta.reshape(1), xc, W_enc, latent_bias.reshape(1, N_DIRS))


def _skey(v):
    bits = jax.lax.bitcast_convert_type(v, jnp.int32)
    return jnp.where(bits >= 0, bits, bits ^ np.int32(0x7FFFFFFF))


def _skey_to_float(sk):
    bits = jnp.where(sk >= 0, sk, sk ^ np.int32(0x7FFFFFFF))
    return jax.lax.bitcast_convert_type(bits, jnp.float32)


def _search_kth(skeys, kth):
    rows = skeys.shape[0]
    ut0 = jnp.zeros((rows, 1), jnp.int32)

    def body(i, ut):
        b = 31 - i
        cand_u = ut | (np.int32(1) << b)
        cand_s = cand_u ^ MINT
        cnt = jnp.sum((skeys >= cand_s).astype(jnp.int32), axis=1, keepdims=True)
        return jnp.where(cnt >= kth, cand_u, ut)

    ut = jax.lax.fori_loop(0, 32, body, ut0)
    return ut ^ MINT


THR_T_TILE = 128


def _thr_body(lpa_ref, t64_ref, t256_ref, fired_ref):
    skeys = _skey(lpa_ref[...])
    t64 = _search_kth(skeys, K)
    t256 = _search_kth(skeys, 4 * K)
    t64_ref[...] = jnp.broadcast_to(_skey_to_float(t64), t64_ref.shape)
    t256_ref[...] = jnp.broadcast_to(_skey_to_float(t256), t256_ref.shape)
    fired = jnp.max((skeys >= t64).astype(jnp.int32), axis=0, keepdims=True)
    fired_ref[...] = fired.reshape(fired_ref.shape)


def _thresholds(lpa):
    grid = (TOKENS // THR_T_TILE,)
    return pl.pallas_call(
        _thr_body,
        grid=grid,
        in_specs=[pl.BlockSpec((THR_T_TILE, N_DIRS), lambda t: (t, 0))],
        out_specs=[
            pl.BlockSpec((THR_T_TILE, 128), lambda t: (t, 0)),
            pl.BlockSpec((THR_T_TILE, 128), lambda t: (t, 0)),
            pl.BlockSpec((1, 1, N_DIRS), lambda t: (t, 0, 0)),
        ],
        out_shape=[
            jax.ShapeDtypeStruct((TOKENS, 128), jnp.float32),
            jax.ShapeDtypeStruct((TOKENS, 128), jnp.float32),
            jax.ShapeDtypeStruct((TOKENS // THR_T_TILE, 1, N_DIRS), jnp.int32),
        ],
    )(lpa)


def _aux_body(lpa_ref, stats_ref, fired_ref, taux_ref, dead_ref):
    fired = jnp.max(fired_ref[...], axis=(0, 1)).reshape(1, N_DIRS)
    alive = stats_ref[...] + 1 <= DEAD_STEPS_THRESHOLD
    dead = jnp.logical_and(jnp.logical_not(alive), fired == 0)
    skeys = jnp.where(dead, _skey(lpa_ref[...]), jnp.zeros((), jnp.int32))
    taux = _search_kth(skeys, AUXK)
    taux_ref[...] = jnp.broadcast_to(_skey_to_float(taux), taux_ref.shape)
    dead_ref[...] = dead.astype(jnp.float32)


def _aux_thresholds(lpa, stats, fired_part):
    grid = (TOKENS // THR_T_TILE,)
    nfp = fired_part.shape[0]
    return pl.pallas_call(
        _aux_body,
        grid=grid,
        in_specs=[
            pl.BlockSpec((THR_T_TILE, N_DIRS), lambda t: (t, 0)),
            pl.BlockSpec((1, N_DIRS), lambda t: (0, 0)),
            pl.BlockSpec((nfp, 1, N_DIRS), lambda t: (0, 0, 0)),
        ],
        out_specs=[
            pl.BlockSpec((THR_T_TILE, 128), lambda t: (t, 0)),
            pl.BlockSpec((1, N_DIRS), lambda t: (0, 0)),
        ],
        out_shape=[
            jax.ShapeDtypeStruct((TOKENS, 128), jnp.float32),
            jax.ShapeDtypeStruct((1, N_DIRS), jnp.float32),
        ],
    )(lpa, stats.reshape(1, N_DIRS), fired_part)


REC_T_TILE = 512
REC_N_TILE = 2048


def _rec_body(lpa_ref, w_ref, t64_ref, t256_ref, pb_ref, r64_ref, r256_ref,
              acc64, acc256):
    n = pl.program_id(1)
    lpa = lpa_ref[...]
    relu = jnp.maximum(lpa, 0.0)
    t64 = t64_ref[...][:, :1]
    t256 = t256_ref[...][:, :1]
    l64 = jnp.where(lpa >= t64, relu, 0.0)
    l256 = jnp.where(lpa >= t256, relu, 0.0)
    w = w_ref[...]
    p64 = jax.lax.dot_general(l64, w, (((1,), (1,)), ((), ())),
                              preferred_element_type=jnp.float32)
    p256 = jax.lax.dot_general(l256, w, (((1,), (1,)), ((), ())),
                               preferred_element_type=jnp.float32)

    @pl.when(n == 0)
    def _init():
        acc64[...] = p64
        acc256[...] = p256

    @pl.when(n != 0)
    def _acc():
        acc64[...] += p64
        acc256[...] += p256

    @pl.when(n == pl.num_programs(1) - 1)
    def _emit():
        pb = pb_ref[...]
        r64_ref[...] = acc64[...] + pb
        r256_ref[...] = acc256[...] + pb


def _recons(lpa, W_dec, t64f, t256f, pre_bias):
    grid = (TOKENS // REC_T_TILE, N_DIRS // REC_N_TILE)
    return pl.pallas_call(
        _rec_body,
        grid=grid,
        in_specs=[
            pl.BlockSpec((REC_T_TILE, REC_N_TILE), lambda t, n: (t, n)),
            pl.BlockSpec((D_MODEL, REC_N_TILE), lambda t, n: (0, n)),
            pl.BlockSpec((REC_T_TILE, 128), lambda t, n: (t, 0)),
            pl.BlockSpec((REC_T_TILE, 128), lambda t, n: (t, 0)),
            pl.BlockSpec((1, D_MODEL), lambda t, n: (0, 0)),
        ],
        out_specs=[
            pl.BlockSpec((REC_T_TILE, D_MODEL), lambda t, n: (t, 0)),
            pl.BlockSpec((REC_T_TILE, D_MODEL), lambda t, n: (t, 0)),
        ],
        out_shape=[
            jax.ShapeDtypeStruct((TOKENS, D_MODEL), jnp.float32),
            jax.ShapeDtypeStruct((TOKENS, D_MODEL), jnp.float32),
        ],
        scratch_shapes=[
            pltpu.VMEM((REC_T_TILE, D_MODEL), jnp.float32),
            pltpu.VMEM((REC_T_TILE, D_MODEL), jnp.float32),
        ],
    )(lpa, W_dec, t64f, t256f, pre_bias.reshape(1, D_MODEL))


def kernel(x, W_enc, W_dec, pre_bias, latent_bias, stats_last_nonzero, theta):
    xc = x - pre_bias
    theta = jnp.asarray(theta, jnp.float32)
    lpa, latents_jump = _encoder(xc, W_enc, latent_bias, theta)

    t64f, t256f, fired_part = _thresholds(lpa)
    tauxf, dead = _aux_thresholds(lpa, stats_last_nonzero, fired_part)
    recons, multik_recons = _recons(lpa, W_dec, t64f, t256f, pre_bias)

    multik_values, multik_indices = lax.approx_max_k(
        lpa, 4 * K, recall_target=1.0, aggregate_to_topk=True)
    multik_values = jnp.maximum(multik_values, 0.0)
    topk_indices = multik_indices[:, :K]
    topk_values = multik_values[:, :K]

    auxk_values, auxk_indices = lax.approx_max_k(
        lpa * dead, AUXK, recall_target=1.0, aggregate_to_topk=True)
    auxk_values = jnp.maximum(auxk_values, 0.0)

    return (recons, topk_indices, topk_values, multik_indices, multik_values,
            multik_recons, auxk_indices, auxk_values, lpa, latents_jump)
